# baseline (speedup 1.0000x reference)
import functools

import jax
import jax.numpy as jnp
from jax import lax
from jax.experimental import pallas as pl
from jax.experimental.pallas import tpu as pltpu
from jax.experimental.pallas import tpu_sc as plsc

B, D, HW = 8, 256, 1024
N = B * HW
K = 1024
BETA = 0.25
_LOSS_SCALE = (1.0 + BETA) / (N * D)

_NC, _NS = 2, 16
_NW = _NC * _NS

_R = 1024
_C = 4
_CH = N // _C
_CH_GRID = _CH // _R
_CHPW = _CH // _NW


def _norm_table(e):
    es = jnp.sum(e * e, axis=1, keepdims=True)
    en = e * (1.0 / jnp.maximum(jnp.sqrt(es), 1e-12))
    entv = -2.0 * en.T
    esq = 0.25 * jnp.sum(entv * entv, axis=0, keepdims=True)
    return en, entv, esq


def _row_block(zr_ref, ent, e_sq, idx_ref, loss_ref):
    zr = zr_ref[...]
    s = jnp.sum(zr * zr, axis=1, keepdims=True)
    inv = 1.0 / jnp.maximum(jnp.sqrt(s), 1e-12)
    zn = zr * inv
    znsq = s * inv * inv

    g = jnp.dot(zn, ent, preferred_element_type=jnp.float32)
    gd = g + e_sq

    minv = jnp.min(gd, axis=1, keepdims=True)
    iota_l = jax.lax.broadcasted_iota(jnp.int32, (_R, K), 1)
    idxm = jnp.min(jnp.where(gd == minv, iota_l, 2 ** 30), axis=1,
                   keepdims=True)
    idx_ref[...] = idxm.T[0]

    loss_ref[0, 0] += jnp.sum(minv + znsq) * _LOSS_SCALE


def _vq_body0(zr_ref, e_ref, idx_ref, loss_ref, en_ref, ent_ref, esq_ref):
    step = pl.program_id(0)

    @pl.when(step == 0)
    def _():
        en, entv, esq = _norm_table(e_ref[...])
        en_ref[...] = en
        ent_ref[...] = entv
        esq_ref[...] = esq
        loss_ref[0, 0] = 0.0

    _row_block(zr_ref, ent_ref[...], esq_ref[...], idx_ref, loss_ref)


def _vq_bodyn(zr_ref, ent_ref, esq_ref, idx_ref, loss_ref):
    @pl.when(pl.program_id(0) == 0)
    def _():
        loss_ref[0, 0] = 0.0

    _row_block(zr_ref, ent_ref[...], esq_ref[...], idx_ref, loss_ref)


def _tc_chunk0(zr, embedding):
    return pl.pallas_call(
        _vq_body0,
        grid=(_CH_GRID,),
        in_specs=[
            pl.BlockSpec((_R, D), lambda i: (i, 0)),
            pl.BlockSpec((K, D), lambda i: (0, 0)),
        ],
        out_specs=[
            pl.BlockSpec((_R,), lambda i: (i,)),
            pl.BlockSpec((1, 1), lambda i: (0, 0), memory_space=pltpu.SMEM),
            pl.BlockSpec((K, D), lambda i: (0, 0)),
            pl.BlockSpec((D, K), lambda i: (0, 0)),
            pl.BlockSpec((1, K), lambda i: (0, 0)),
        ],
        out_shape=[
            jax.ShapeDtypeStruct((_CH,), jnp.int32),
            jax.ShapeDtypeStruct((1, 1), jnp.float32),
            jax.ShapeDtypeStruct((K, D), jnp.float32),
            jax.ShapeDtypeStruct((D, K), jnp.float32),
            jax.ShapeDtypeStruct((1, K), jnp.float32),
        ],
    )(zr, embedding)


def _make_tc_chunk(c):
    def run(zr, ent, esq):
        return pl.pallas_call(
            _vq_bodyn,
            grid=(_CH_GRID,),
            in_specs=[
                pl.BlockSpec((_R, D), lambda i: (c * _CH_GRID + i, 0)),
                pl.BlockSpec((D, K), lambda i: (0, 0)),
                pl.BlockSpec((1, K), lambda i: (0, 0)),
            ],
            out_specs=[
                pl.BlockSpec((_R,), lambda i: (i,)),
                pl.BlockSpec((1, 1), lambda i: (0, 0),
                             memory_space=pltpu.SMEM),
            ],
            out_shape=[
                jax.ShapeDtypeStruct((_CH,), jnp.int32),
                jax.ShapeDtypeStruct((1, 1), jnp.float32),
            ],
        )(zr, ent, esq)
    return run


_TC_CHUNKS = [_make_tc_chunk(c) for c in range(1, _C)]

_SC_MESH = plsc.VectorSubcoreMesh(core_axis_name="c", subcore_axis_name="s")


def _make_sc_chunk(c):
    @functools.partial(
        pl.kernel,
        mesh=_SC_MESH,
        compiler_params=pltpu.CompilerParams(needs_layout_passes=False),
        scratch_types=[
            pltpu.VMEM((_CHPW,), jnp.int32),
            pltpu.VMEM((_CHPW, D), jnp.float32),
            pltpu.SemaphoreType.DMA,
        ],
    )
    def gather(en_hbm, idx_hbm, out_ref, idx_v, rows_v, sem):
        wid = lax.axis_index("s") * _NC + lax.axis_index("c")
        base = wid * _CHPW
        pltpu.sync_copy(idx_hbm.at[pl.ds(base, _CHPW)], idx_v)
        pltpu.async_copy(en_hbm.at[idx_v], rows_v, sem).wait()
        pltpu.sync_copy(rows_v, out_ref.at[pl.ds(c * _CH + base, _CHPW)])
    return gather


_SC_CHUNKS = [_make_sc_chunk(c) for c in range(_C)]


def kernel(z, embedding):
    zr = jnp.transpose(z, (0, 2, 3, 1)).reshape(N, D)
    zq_ref = jax.empty_ref(jax.ShapeDtypeStruct((N, D), jnp.float32))

    idx0, loss, en, ent, esq = _tc_chunk0(zr, embedding)
    _SC_CHUNKS[0](en, idx0, zq_ref)
    idxs = [idx0]
    for c in range(1, _C):
        idxc, lossc = _TC_CHUNKS[c - 1](zr, ent, esq)
        _SC_CHUNKS[c](en, idxc, zq_ref)
        idxs.append(idxc)
        loss = loss + lossc

    zq_rows = zq_ref[...]
    zq = jnp.transpose(zq_rows.reshape(B, 32, 32, D), (0, 3, 1, 2))
    return (zq, jnp.concatenate(idxs), loss[0, 0])


# speedup vs baseline: 1.1560x; 1.1560x over previous
import functools

import jax
import jax.numpy as jnp
from jax import lax
from jax.experimental import pallas as pl
from jax.experimental.pallas import tpu as pltpu
from jax.experimental.pallas import tpu_sc as plsc

B, D, HW = 8, 256, 1024
N = B * HW
K = 1024
BETA = 0.25
_LOSS_SCALE = (1.0 + BETA) / (N * D)

_NC, _NS = 2, 16
_NW = _NC * _NS
_RPW = N // _NW
_ICH = 128
_NI = _RPW // _ICH

_R = 4096
_GRID = N // _R


def _vq_body(zr_ref, e_ref, idx_ref, loss_ref, en_ref, ent_s, esq_s):
    step = pl.program_id(0)

    @pl.when(step == 0)
    def _():
        e = e_ref[...]
        es = jnp.sum(e * e, axis=1, keepdims=True)
        en = e * (1.0 / jnp.maximum(jnp.sqrt(es), 1e-12))
        en_ref[...] = en
        entv = -2.0 * en.T
        ent_s[...] = entv
        esq_s[...] = 0.25 * jnp.sum(entv * entv, axis=0, keepdims=True)
        loss_ref[0, 0] = 0.0

    ent = ent_s[...]
    e_sq = esq_s[...]

    zr = zr_ref[...]
    s = jnp.sum(zr * zr, axis=1, keepdims=True)
    inv = 1.0 / jnp.maximum(jnp.sqrt(s), 1e-12)
    zn = zr * inv
    znsq = s * inv * inv

    g = jnp.dot(zn, ent, preferred_element_type=jnp.float32)
    gd = g + e_sq

    minv = jnp.min(gd, axis=1, keepdims=True)
    iota_l = jax.lax.broadcasted_iota(jnp.int32, (_R, K), 1)
    idxm = jnp.min(jnp.where(gd == minv, iota_l, 2 ** 30), axis=1,
                   keepdims=True)
    idx_ref[...] = idxm.T[0]

    loss_ref[0, 0] += jnp.sum(minv + znsq) * _LOSS_SCALE


def _tc_stage(zr, embedding):
    return pl.pallas_call(
        _vq_body,
        grid=(_GRID,),
        in_specs=[
            pl.BlockSpec((_R, D), lambda i: (i, 0)),
            pl.BlockSpec((K, D), lambda i: (0, 0)),
        ],
        out_specs=[
            pl.BlockSpec((_R,), lambda i: (i,)),
            pl.BlockSpec((1, 1), lambda i: (0, 0), memory_space=pltpu.SMEM),
            pl.BlockSpec((K, D), lambda i: (0, 0)),
        ],
        out_shape=[
            jax.ShapeDtypeStruct((N,), jnp.int32),
            jax.ShapeDtypeStruct((1, 1), jnp.float32),
            jax.ShapeDtypeStruct((K, D), jnp.float32),
        ],
        scratch_shapes=[pltpu.VMEM((D, K), jnp.float32),
                        pltpu.VMEM((1, K), jnp.float32)],
    )(zr, embedding)


@functools.partial(
    pl.kernel,
    out_type=jax.ShapeDtypeStruct((N, D), jnp.float32),
    mesh=plsc.VectorSubcoreMesh(core_axis_name="c", subcore_axis_name="s"),
    compiler_params=pltpu.CompilerParams(needs_layout_passes=False),
    scratch_types=[
        pltpu.VMEM((_NI, _ICH), jnp.int32),
        pltpu.VMEM((_RPW, D), jnp.float32),
        pltpu.SemaphoreType.DMA,
    ],
)
def _sc_gather(en_hbm, idx2_hbm, out_hbm, idx_v, rows_v, sem):
    wid = lax.axis_index("s") * _NC + lax.axis_index("c")
    pltpu.sync_copy(idx2_hbm.at[pl.ds(wid * _NI, _NI)], idx_v)
    copies = [
        pltpu.async_copy(en_hbm.at[idx_v.at[j]],
                         rows_v.at[pl.ds(j * _ICH, _ICH)], sem)
        for j in range(_NI)
    ]
    for j, cp in enumerate(copies):
        cp.wait()
        pltpu.sync_copy(rows_v.at[pl.ds(j * _ICH, _ICH)],
                        out_hbm.at[pl.ds(wid * _RPW + j * _ICH, _ICH)])


def kernel(z, embedding):
    zr = jnp.transpose(z, (0, 2, 3, 1)).reshape(N, D)
    idx, loss, en = _tc_stage(zr, embedding)
    zq_rows = _sc_gather(en, idx.reshape(N // _ICH, _ICH))
    zq = jnp.transpose(zq_rows.reshape(B, 32, 32, D), (0, 3, 1, 2))
    return (zq, idx, loss[0, 0])
